# Initial kernel scaffold; baseline (speedup 1.0000x reference)
import functools

import jax
import jax.numpy as jnp
from jax import lax
from jax.experimental import pallas as pl
from jax.experimental.pallas import tpu as pltpu
from jax.experimental.pallas import tpu_sc as plsc

N_NODES = 10000
N_EDGES = 320000
D_FEAT = 128
D_EDGE = 16

NC, NS = 2, 16
NW = NC * NS
CHUNK = 128
N_CHUNKS = N_EDGES // CHUNK
GATHER_CPW = -(-N_CHUNKS // NW)
SCAT_CPS = -(-N_CHUNKS // NS)
ROWS_PER_SUB = N_NODES // NS
ZROWS = 125

_VMESH = plsc.VectorSubcoreMesh(
    core_axis_name="c", subcore_axis_name="s", num_cores=NC, num_subcores=NS)


def _premix_body(xw_ref, xm_ref, ww_ref, wm_ref,
                 sw_ref, dw_ref, sm_ref, dm_ref):
    pw = jnp.dot(xw_ref[...], ww_ref[...], preferred_element_type=jnp.float32)
    pm = jnp.dot(xm_ref[...], wm_ref[...], preferred_element_type=jnp.float32)
    sw_ref[...] = pw[:, :D_FEAT]
    dw_ref[...] = pw[:, D_FEAT:]
    sm_ref[...] = pm[:, :D_FEAT]
    dm_ref[...] = pm[:, D_FEAT:]


def _premix(x_w, x_m, wcat_w, wcat_m):
    n_tile = 1000
    grid = (N_NODES // n_tile,)
    node_spec = pl.BlockSpec((n_tile, D_FEAT), lambda t: (t, 0))
    w_spec = pl.BlockSpec((D_FEAT, 2 * D_FEAT), lambda t: (0, 0))
    out = jax.ShapeDtypeStruct((N_NODES, D_FEAT), jnp.float32)
    return pl.pallas_call(
        _premix_body,
        grid=grid,
        in_specs=[node_spec, node_spec, w_spec, w_spec],
        out_specs=[node_spec] * 4,
        out_shape=[out] * 4,
    )(x_w, x_m, wcat_w, wcat_m)


@functools.partial(
    pl.kernel,
    out_type=[jax.ShapeDtypeStruct((N_EDGES, D_FEAT), jnp.float32),
              jax.ShapeDtypeStruct((N_EDGES, D_FEAT), jnp.float32)],
    mesh=_VMESH,
    scratch_types=[
        pltpu.VMEM((CHUNK,), jnp.int32),
        pltpu.VMEM((CHUNK,), jnp.int32),
        pltpu.VMEM((CHUNK, D_FEAT), jnp.float32),
        pltpu.VMEM((CHUNK, D_FEAT), jnp.float32),
        pltpu.SemaphoreType.DMA,
        pltpu.SemaphoreType.DMA,
    ],
)
def _sc_gather(sw_hbm, dw_hbm, sm_hbm, dm_hbm,
               srcw_hbm, dstw_hbm, srcm_hbm, dstm_hbm,
               gw_hbm, gm_hbm,
               idx_s, idx_d, sbuf, dbuf, sem_s, sem_d):
    wid = lax.axis_index("s") * NC + lax.axis_index("c")

    def do_model(s_tab, d_tab, src_hbm, dst_hbm, out_hbm):
        @pl.loop(0, GATHER_CPW)
        def _(j):
            chunk = wid + NW * j

            @pl.when(chunk < N_CHUNKS)
            def _():
                base = chunk * CHUNK
                pltpu.sync_copy(src_hbm.at[pl.ds(base, CHUNK)], idx_s)
                pltpu.sync_copy(dst_hbm.at[pl.ds(base, CHUNK)], idx_d)
                cs = pltpu.async_copy(s_tab.at[idx_s], sbuf, sem_s)
                cd = pltpu.async_copy(d_tab.at[idx_d], dbuf, sem_d)
                cs.wait()
                cd.wait()

                @pl.loop(0, CHUNK)
                def _(r):
                    @pl.loop(0, D_FEAT, step=16)
                    def _(cc):
                        sl = pl.ds(cc, 16)
                        sbuf[r, sl] = sbuf[r, sl] + dbuf[r, sl]

                pltpu.sync_copy(sbuf, out_hbm.at[pl.ds(base, CHUNK)])

    do_model(sw_hbm, dw_hbm, srcw_hbm, dstw_hbm, gw_hbm)
    do_model(sm_hbm, dm_hbm, srcm_hbm, dstm_hbm, gm_hbm)


def _edge_body(gw_ref, aw_ref, gm_ref, am_ref,
               cw_ref, b1w_ref, w2w_ref, b2w_ref,
               cm_ref, b1m_ref, w2m_ref, b2m_ref,
               ew_ref, em_ref):
    hw = (gw_ref[...]
          + jnp.dot(aw_ref[...], cw_ref[...],
                    preferred_element_type=jnp.float32) + b1w_ref[...])
    hw = jnp.maximum(hw, 0.0)
    ew_ref[...] = (jnp.dot(hw, w2w_ref[...],
                           preferred_element_type=jnp.float32) + b2w_ref[...])
    hm = (gm_ref[...]
          + jnp.dot(am_ref[...], cm_ref[...],
                    preferred_element_type=jnp.float32) + b1m_ref[...])
    hm = jnp.maximum(hm, 0.0)
    em_ref[...] = (jnp.dot(hm, w2m_ref[...],
                           preferred_element_type=jnp.float32) + b2m_ref[...])


def _edge_mlp(gw, attrw, gm, attrm, cw, b1w, w2w, b2w, cm, b1m, w2m, b2m):
    e_tile = 512
    grid = (N_EDGES // e_tile,)
    g_spec = pl.BlockSpec((e_tile, D_FEAT), lambda t: (t, 0))
    a_spec = pl.BlockSpec((e_tile, D_EDGE), lambda t: (t, 0))
    c_spec = pl.BlockSpec((D_EDGE, D_FEAT), lambda t: (0, 0))
    w_spec = pl.BlockSpec((D_FEAT, D_FEAT), lambda t: (0, 0))
    b_spec = pl.BlockSpec((1, D_FEAT), lambda t: (0, 0))
    out = jax.ShapeDtypeStruct((N_EDGES, D_FEAT), jnp.float32)
    return pl.pallas_call(
        _edge_body,
        grid=grid,
        in_specs=[g_spec, a_spec, g_spec, a_spec,
                  c_spec, b_spec, w_spec, b_spec,
                  c_spec, b_spec, w_spec, b_spec],
        out_specs=[g_spec, g_spec],
        out_shape=[out, out],
    )(gw, attrw, gm, attrm, cw, b1w, w2w, b2w, cm, b1m, w2m, b2m)


@functools.partial(
    pl.kernel,
    out_type=[jax.ShapeDtypeStruct((N_NODES, D_FEAT), jnp.float32),
              jax.ShapeDtypeStruct((N_NODES, D_FEAT), jnp.float32)],
    mesh=_VMESH,
    scratch_types=[
        pltpu.VMEM_SHARED((N_NODES, D_FEAT), jnp.float32),
        pltpu.VMEM((ZROWS, D_FEAT), jnp.float32),
        pltpu.VMEM((CHUNK,), jnp.int32),
        pltpu.VMEM((CHUNK, D_FEAT), jnp.float32),
    ],
)
def _sc_segsum(ew_hbm, em_hbm, dstw_hbm, dstm_hbm,
               aw_hbm, am_hbm,
               agg_sh, zbuf, idx_v, ebuf):
    cid = lax.axis_index("c")
    sid = lax.axis_index("s")

    @pl.loop(0, ZROWS)
    def _(r):
        @pl.loop(0, D_FEAT, step=16)
        def _(cc):
            zbuf[r, pl.ds(cc, 16)] = jnp.zeros((16,), jnp.float32)

    @pl.loop(0, ROWS_PER_SUB // ZROWS)
    def _(k):
        pltpu.sync_copy(
            zbuf, agg_sh.at[pl.ds(sid * ROWS_PER_SUB + k * ZROWS, ZROWS)])

    plsc.subcore_barrier()

    def accumulate(e_hbm, dst_hbm):
        @pl.loop(0, SCAT_CPS)
        def _(j):
            chunk = sid + NS * j

            @pl.when(chunk < N_CHUNKS)
            def _():
                base = chunk * CHUNK
                pltpu.sync_copy(dst_hbm.at[pl.ds(base, CHUNK)], idx_v)
                pltpu.sync_copy(e_hbm.at[pl.ds(base, CHUNK)], ebuf)
                pltpu.sync_copy(ebuf, agg_sh.at[idx_v], add=True)

    @pl.when(cid == 0)
    def _():
        accumulate(ew_hbm, dstw_hbm)

    @pl.when(cid == 1)
    def _():
        accumulate(em_hbm, dstm_hbm)

    plsc.subcore_barrier()

    row_sl = pl.ds(sid * ROWS_PER_SUB, ROWS_PER_SUB)

    @pl.when(cid == 0)
    def _():
        pltpu.sync_copy(agg_sh.at[row_sl], aw_hbm.at[row_sl])

    @pl.when(cid == 1)
    def _():
        pltpu.sync_copy(agg_sh.at[row_sl], am_hbm.at[row_sl])


def _node_body(x_ref, aw_ref, am_ref, wn1_ref, bn1_ref, wn2_ref, bn2_ref,
               out_ref):
    hn = jnp.concatenate([x_ref[...], aw_ref[...], am_ref[...]], axis=1)
    h = jnp.dot(hn, wn1_ref[...], preferred_element_type=jnp.float32)
    h = jnp.maximum(h + bn1_ref[...], 0.0)
    out_ref[...] = (jnp.dot(h, wn2_ref[...],
                            preferred_element_type=jnp.float32) + bn2_ref[...])


def _node_mlp(x_m, aggw, aggm, wn1, bn1, wn2, bn2):
    n_tile = 1000
    grid = (N_NODES // n_tile,)
    node_spec = pl.BlockSpec((n_tile, D_FEAT), lambda t: (t, 0))
    wn1_spec = pl.BlockSpec((3 * D_FEAT, D_FEAT), lambda t: (0, 0))
    w_spec = pl.BlockSpec((D_FEAT, D_FEAT), lambda t: (0, 0))
    b_spec = pl.BlockSpec((1, D_FEAT), lambda t: (0, 0))
    out = jax.ShapeDtypeStruct((N_NODES, D_FEAT), jnp.float32)
    return pl.pallas_call(
        _node_body,
        grid=grid,
        in_specs=[node_spec, node_spec, node_spec,
                  wn1_spec, b_spec, w_spec, b_spec],
        out_specs=node_spec,
        out_shape=out,
    )(x_m, aggw, aggm, wn1, bn1, wn2, bn2)


def kernel(x_m, x_w, edge_w, edge_m, edge_attrw, edge_attrm,
           W1w, b1w, W2w, b2w, W1m, b1m, W2m, b2m,
           Wn1, bn1, Wn2, bn2):
    srcw = edge_w[0].astype(jnp.int32)
    dstw = edge_w[1].astype(jnp.int32)
    srcm = edge_m[0].astype(jnp.int32)
    dstm = edge_m[1].astype(jnp.int32)

    wcat_w = jnp.concatenate([W1w[:D_FEAT], W1w[D_FEAT:2 * D_FEAT]], axis=1)
    wcat_m = jnp.concatenate([W1m[:D_FEAT], W1m[D_FEAT:2 * D_FEAT]], axis=1)
    cw = W1w[2 * D_FEAT:]
    cm = W1m[2 * D_FEAT:]

    sw, dw, sm, dm = _premix(x_w, x_m, wcat_w, wcat_m)

    gw, gm = _sc_gather(sw, dw, sm, dm, srcw, dstw, srcm, dstm)

    ew, em = _edge_mlp(gw, edge_attrw, gm, edge_attrm,
                       cw, b1w.reshape(1, -1), W2w, b2w.reshape(1, -1),
                       cm, b1m.reshape(1, -1), W2m, b2m.reshape(1, -1))

    aggw, aggm = _sc_segsum(ew, em, dstw, dstm)

    x = _node_mlp(x_m, aggw, aggm, Wn1,
                  bn1.reshape(1, -1), Wn2, bn2.reshape(1, -1))
    return (x, ew, em)


# speedup vs baseline: 2.4500x; 2.4500x over previous
import functools

import jax
import jax.numpy as jnp
from jax import lax
from jax.experimental import pallas as pl
from jax.experimental.pallas import tpu as pltpu
from jax.experimental.pallas import tpu_sc as plsc

N_NODES = 10000
N_EDGES = 320000
D_FEAT = 128
D_EDGE = 16

NC, NS = 2, 16
NW = NC * NS
CHUNK = 128
N_CHUNKS = N_EDGES // CHUNK
GATHER_CPW = -(-N_CHUNKS // NW)
SCAT_CPS = -(-N_CHUNKS // NS)
ROW_CHUNK = 200
N_ROW_CHUNKS = N_NODES // ROW_CHUNK
ROW_CPS = -(-N_ROW_CHUNKS // NS)

_VMESH = plsc.VectorSubcoreMesh(
    core_axis_name="c", subcore_axis_name="s", num_cores=NC, num_subcores=NS)


def _premix_body(xw_ref, xm_ref, ww_ref, wm_ref,
                 sw_ref, dw_ref, sm_ref, dm_ref):
    pw = jnp.dot(xw_ref[...], ww_ref[...], preferred_element_type=jnp.float32)
    pm = jnp.dot(xm_ref[...], wm_ref[...], preferred_element_type=jnp.float32)
    sw_ref[...] = pw[:, :D_FEAT]
    dw_ref[...] = pw[:, D_FEAT:]
    sm_ref[...] = pm[:, :D_FEAT]
    dm_ref[...] = pm[:, D_FEAT:]


def _premix(x_w, x_m, wcat_w, wcat_m):
    n_tile = 1000
    grid = (N_NODES // n_tile,)
    node_spec = pl.BlockSpec((n_tile, D_FEAT), lambda t: (t, 0))
    w_spec = pl.BlockSpec((D_FEAT, 2 * D_FEAT), lambda t: (0, 0))
    out = jax.ShapeDtypeStruct((N_NODES, D_FEAT), jnp.float32)
    return pl.pallas_call(
        _premix_body,
        grid=grid,
        in_specs=[node_spec, node_spec, w_spec, w_spec],
        out_specs=[node_spec] * 4,
        out_shape=[out] * 4,
    )(x_w, x_m, wcat_w, wcat_m)


@functools.partial(
    pl.kernel,
    out_type=[jax.ShapeDtypeStruct((N_EDGES, D_FEAT), jnp.float32),
              jax.ShapeDtypeStruct((N_EDGES, D_FEAT), jnp.float32)],
    mesh=_VMESH,
    scratch_types=[
        pltpu.VMEM((CHUNK,), jnp.int32),
        pltpu.VMEM((CHUNK,), jnp.int32),
        pltpu.VMEM((CHUNK, D_FEAT), jnp.float32),
        pltpu.VMEM((CHUNK, D_FEAT), jnp.float32),
        pltpu.SemaphoreType.DMA,
        pltpu.SemaphoreType.DMA,
    ],
)
def _sc_gather(sw_hbm, dw_hbm, sm_hbm, dm_hbm,
               srcw_hbm, dstw_hbm, srcm_hbm, dstm_hbm,
               gw_hbm, gm_hbm,
               idx_s, idx_d, sbuf, dbuf, sem_s, sem_d):
    wid = lax.axis_index("s") * NC + lax.axis_index("c")

    def do_model(s_tab, d_tab, src_hbm, dst_hbm, out_hbm):
        @pl.loop(0, GATHER_CPW)
        def _(j):
            chunk = wid + NW * j

            @pl.when(chunk < N_CHUNKS)
            def _():
                base = pl.multiple_of(chunk * CHUNK, 8)
                pltpu.sync_copy(src_hbm.at[pl.ds(base, CHUNK)], idx_s)
                pltpu.sync_copy(dst_hbm.at[pl.ds(base, CHUNK)], idx_d)
                cs = pltpu.async_copy(s_tab.at[idx_s], sbuf, sem_s)
                cd = pltpu.async_copy(d_tab.at[idx_d], dbuf, sem_d)
                cs.wait()
                cd.wait()

                @pl.loop(0, CHUNK)
                def _(r):
                    @pl.loop(0, D_FEAT, step=16)
                    def _(cc):
                        sl = pl.ds(cc, 16)
                        sbuf[r, sl] = sbuf[r, sl] + dbuf[r, sl]

                pltpu.sync_copy(sbuf, out_hbm.at[pl.ds(base, CHUNK)])

    do_model(sw_hbm, dw_hbm, srcw_hbm, dstw_hbm, gw_hbm)
    do_model(sm_hbm, dm_hbm, srcm_hbm, dstm_hbm, gm_hbm)


def _edge_body(gw_ref, aw_ref, gm_ref, am_ref,
               cw_ref, b1w_ref, w2w_ref, b2w_ref,
               cm_ref, b1m_ref, w2m_ref, b2m_ref,
               ew_ref, em_ref):
    hw = (gw_ref[...]
          + jnp.dot(aw_ref[...], cw_ref[...],
                    preferred_element_type=jnp.float32) + b1w_ref[...])
    hw = jnp.maximum(hw, 0.0)
    ew_ref[...] = (jnp.dot(hw, w2w_ref[...],
                           preferred_element_type=jnp.float32) + b2w_ref[...])
    hm = (gm_ref[...]
          + jnp.dot(am_ref[...], cm_ref[...],
                    preferred_element_type=jnp.float32) + b1m_ref[...])
    hm = jnp.maximum(hm, 0.0)
    em_ref[...] = (jnp.dot(hm, w2m_ref[...],
                           preferred_element_type=jnp.float32) + b2m_ref[...])


def _edge_mlp(gw, attrw, gm, attrm, cw, b1w, w2w, b2w, cm, b1m, w2m, b2m):
    e_tile = 512
    grid = (N_EDGES // e_tile,)
    g_spec = pl.BlockSpec((e_tile, D_FEAT), lambda t: (t, 0))
    a_spec = pl.BlockSpec((e_tile, D_EDGE), lambda t: (t, 0))
    c_spec = pl.BlockSpec((D_EDGE, D_FEAT), lambda t: (0, 0))
    w_spec = pl.BlockSpec((D_FEAT, D_FEAT), lambda t: (0, 0))
    b_spec = pl.BlockSpec((1, D_FEAT), lambda t: (0, 0))
    out = jax.ShapeDtypeStruct((N_EDGES, D_FEAT), jnp.float32)
    return pl.pallas_call(
        _edge_body,
        grid=grid,
        in_specs=[g_spec, a_spec, g_spec, a_spec,
                  c_spec, b_spec, w_spec, b_spec,
                  c_spec, b_spec, w_spec, b_spec],
        out_specs=[g_spec, g_spec],
        out_shape=[out, out],
    )(gw, attrw, gm, attrm, cw, b1w, w2w, b2w, cm, b1m, w2m, b2m)


@functools.partial(
    pl.kernel,
    out_type=[jax.ShapeDtypeStruct((N_NODES, D_FEAT), jnp.float32),
              jax.ShapeDtypeStruct((N_NODES, D_FEAT), jnp.float32)],
    mesh=_VMESH,
    scratch_types=[
        pltpu.VMEM_SHARED((N_NODES, D_FEAT), jnp.float32),
        pltpu.VMEM((ROW_CHUNK, D_FEAT), jnp.float32),
        pltpu.VMEM((CHUNK,), jnp.int32),
        pltpu.VMEM((CHUNK, D_FEAT), jnp.float32),
    ],
)
def _sc_segsum(ew_hbm, em_hbm, dstw_hbm, dstm_hbm,
               aw_hbm, am_hbm,
               agg_sh, zbuf, idx_v, ebuf):
    cid = lax.axis_index("c")
    sid = lax.axis_index("s")

    @pl.loop(0, ROW_CHUNK)
    def _(r):
        @pl.loop(0, D_FEAT, step=16)
        def _(cc):
            zbuf[r, pl.ds(cc, 16)] = jnp.zeros((16,), jnp.float32)

    @pl.loop(0, ROW_CPS)
    def _(k):
        rchunk = sid + NS * k

        @pl.when(rchunk < N_ROW_CHUNKS)
        def _():
            rbase = pl.multiple_of(rchunk * ROW_CHUNK, 8)
            pltpu.sync_copy(zbuf, agg_sh.at[pl.ds(rbase, ROW_CHUNK)])

    plsc.subcore_barrier()

    def accumulate(e_hbm, dst_hbm):
        @pl.loop(0, SCAT_CPS)
        def _(j):
            chunk = sid + NS * j

            @pl.when(chunk < N_CHUNKS)
            def _():
                base = pl.multiple_of(chunk * CHUNK, 8)
                pltpu.sync_copy(dst_hbm.at[pl.ds(base, CHUNK)], idx_v)
                pltpu.sync_copy(e_hbm.at[pl.ds(base, CHUNK)], ebuf)
                pltpu.sync_copy(ebuf, agg_sh.at[idx_v], add=True)

    @pl.when(cid == 0)
    def _():
        accumulate(ew_hbm, dstw_hbm)

    @pl.when(cid == 1)
    def _():
        accumulate(em_hbm, dstm_hbm)

    plsc.subcore_barrier()

    def copy_out(out_hbm):
        @pl.loop(0, ROW_CPS)
        def _(k):
            rchunk = sid + NS * k

            @pl.when(rchunk < N_ROW_CHUNKS)
            def _():
                rbase = pl.multiple_of(rchunk * ROW_CHUNK, 8)
                sl = pl.ds(rbase, ROW_CHUNK)
                pltpu.sync_copy(agg_sh.at[sl], out_hbm.at[sl])

    @pl.when(cid == 0)
    def _():
        copy_out(aw_hbm)

    @pl.when(cid == 1)
    def _():
        copy_out(am_hbm)


def _node_body(x_ref, aw_ref, am_ref, wn1_ref, bn1_ref, wn2_ref, bn2_ref,
               out_ref):
    hn = jnp.concatenate([x_ref[...], aw_ref[...], am_ref[...]], axis=1)
    h = jnp.dot(hn, wn1_ref[...], preferred_element_type=jnp.float32)
    h = jnp.maximum(h + bn1_ref[...], 0.0)
    out_ref[...] = (jnp.dot(h, wn2_ref[...],
                            preferred_element_type=jnp.float32) + bn2_ref[...])


def _node_mlp(x_m, aggw, aggm, wn1, bn1, wn2, bn2):
    n_tile = 1000
    grid = (N_NODES // n_tile,)
    node_spec = pl.BlockSpec((n_tile, D_FEAT), lambda t: (t, 0))
    wn1_spec = pl.BlockSpec((3 * D_FEAT, D_FEAT), lambda t: (0, 0))
    w_spec = pl.BlockSpec((D_FEAT, D_FEAT), lambda t: (0, 0))
    b_spec = pl.BlockSpec((1, D_FEAT), lambda t: (0, 0))
    out = jax.ShapeDtypeStruct((N_NODES, D_FEAT), jnp.float32)
    return pl.pallas_call(
        _node_body,
        grid=grid,
        in_specs=[node_spec, node_spec, node_spec,
                  wn1_spec, b_spec, w_spec, b_spec],
        out_specs=node_spec,
        out_shape=out,
    )(x_m, aggw, aggm, wn1, bn1, wn2, bn2)


def kernel(x_m, x_w, edge_w, edge_m, edge_attrw, edge_attrm,
           W1w, b1w, W2w, b2w, W1m, b1m, W2m, b2m,
           Wn1, bn1, Wn2, bn2):
    srcw = edge_w[0].astype(jnp.int32)
    dstw = edge_w[1].astype(jnp.int32)
    srcm = edge_m[0].astype(jnp.int32)
    dstm = edge_m[1].astype(jnp.int32)

    wcat_w = jnp.concatenate([W1w[:D_FEAT], W1w[D_FEAT:2 * D_FEAT]], axis=1)
    wcat_m = jnp.concatenate([W1m[:D_FEAT], W1m[D_FEAT:2 * D_FEAT]], axis=1)
    cw = W1w[2 * D_FEAT:]
    cm = W1m[2 * D_FEAT:]

    sw, dw, sm, dm = _premix(x_w, x_m, wcat_w, wcat_m)

    gw, gm = _sc_gather(sw, dw, sm, dm, srcw, dstw, srcm, dstm)

    ew, em = _edge_mlp(gw, edge_attrw, gm, edge_attrm,
                       cw, b1w.reshape(1, -1), W2w, b2w.reshape(1, -1),
                       cm, b1m.reshape(1, -1), W2m, b2m.reshape(1, -1))

    aggw, aggm = _sc_segsum(ew, em, dstw, dstm)

    x = _node_mlp(x_m, aggw, aggm, Wn1,
                  bn1.reshape(1, -1), Wn2, bn2.reshape(1, -1))
    return (x, ew, em)


# speedup vs baseline: 2.8809x; 1.1759x over previous
import functools

import jax
import jax.numpy as jnp
from jax import lax
from jax.experimental import pallas as pl
from jax.experimental.pallas import tpu as pltpu
from jax.experimental.pallas import tpu_sc as plsc

N_NODES = 10000
N_EDGES = 320000
D_FEAT = 128
D_EDGE = 16

NC, NS = 2, 16
NW = NC * NS
CHUNK = 128
N_CHUNKS = N_EDGES // CHUNK
SCAT_CPS = -(-N_CHUNKS // NS)

G_CHUNKS = 2 * N_CHUNKS
G_CPW = -(-G_CHUNKS // NW)
G_CPW_PAD = G_CPW + (G_CPW % 2)
IDX_SPAN = G_CPW_PAD * CHUNK
IDX_PAD = (NW - 1) * G_CPW * CHUNK + IDX_SPAN
ROW_CHUNK = 200
N_ROW_CHUNKS = N_NODES // ROW_CHUNK
ROW_CPS = -(-N_ROW_CHUNKS // NS)

_VMESH = plsc.VectorSubcoreMesh(
    core_axis_name="c", subcore_axis_name="s", num_cores=NC, num_subcores=NS)


def _premix_body(x_ref, w_ref, s_ref, d_ref):
    p = jnp.dot(x_ref[...], w_ref[0], preferred_element_type=jnp.float32)
    s_ref[...] = p[:, :D_FEAT]
    d_ref[...] = p[:, D_FEAT:]


def _premix(x_cat, w_stack):
    n_tile = 1000
    grid = (2 * N_NODES // n_tile,)
    node_spec = pl.BlockSpec((n_tile, D_FEAT), lambda t: (t, 0))
    w_spec = pl.BlockSpec((1, D_FEAT, 2 * D_FEAT), lambda t: (t // 10, 0, 0))
    out = jax.ShapeDtypeStruct((2 * N_NODES, D_FEAT), jnp.float32)
    return pl.pallas_call(
        _premix_body,
        grid=grid,
        in_specs=[node_spec, w_spec],
        out_specs=[node_spec] * 2,
        out_shape=[out] * 2,
    )(x_cat, w_stack)


@functools.partial(
    pl.kernel,
    out_type=jax.ShapeDtypeStruct((2 * N_EDGES, D_FEAT), jnp.float32),
    mesh=_VMESH,
    scratch_types=[
        pltpu.VMEM((IDX_SPAN,), jnp.int32),
        pltpu.VMEM((IDX_SPAN,), jnp.int32),
        [pltpu.VMEM((CHUNK, D_FEAT), jnp.float32)] * 2,
        [pltpu.VMEM((CHUNK, D_FEAT), jnp.float32)] * 2,
        [pltpu.SemaphoreType.DMA] * 2,
        [pltpu.SemaphoreType.DMA] * 2,
    ],
)
def _sc_gather(s_tab, d_tab, isrc_hbm, idst_hbm, out_hbm,
               ibuf_s, ibuf_d, sbufs, dbufs, sems_g, sems_w):
    wid = lax.axis_index("s") * NC + lax.axis_index("c")
    span0 = pl.multiple_of(wid * (G_CPW * CHUNK), 8)

    pltpu.sync_copy(isrc_hbm.at[pl.ds(span0, IDX_SPAN)], ibuf_s)
    pltpu.sync_copy(idst_hbm.at[pl.ds(span0, IDX_SPAN)], ibuf_d)

    def rvalid(j):
        return jnp.logical_and(j < G_CPW, wid * G_CPW + j < G_CHUNKS)

    def issue_gather(j, b):
        isl = pl.ds(pl.multiple_of(j * CHUNK, 8), CHUNK)
        pltpu.async_copy(s_tab.at[ibuf_s.at[isl]], sbufs[b], sems_g[b])
        pltpu.async_copy(d_tab.at[ibuf_d.at[isl]], dbufs[b], sems_g[b])

    def drain_gather(j, b):
        isl = pl.ds(pl.multiple_of(j * CHUNK, 8), CHUNK)
        pltpu.make_async_copy(s_tab.at[ibuf_s.at[isl]], sbufs[b],
                              sems_g[b]).wait()
        pltpu.make_async_copy(d_tab.at[ibuf_d.at[isl]], dbufs[b],
                              sems_g[b]).wait()

    def out_slice(j):
        base = pl.multiple_of((wid * G_CPW + j) * CHUNK, 8)
        return out_hbm.at[pl.ds(base, CHUNK)]

    issue_gather(0, 0)

    def step(j, b):
        @pl.when(rvalid(j))
        def _():
            drain_gather(j, b)

            @pl.when(jnp.logical_and(j >= 1, rvalid(j - 1)))
            def _():
                pltpu.make_async_copy(dbufs[1 - b], out_slice(j - 1),
                                      sems_w[1 - b]).wait()

            @pl.when(rvalid(j + 1))
            def _():
                issue_gather(j + 1, 1 - b)

            @pl.loop(0, CHUNK)
            def _(r):
                for cc in range(0, D_FEAT, 16):
                    sl = pl.ds(cc, 16)
                    dbufs[b][r, sl] = sbufs[b][r, sl] + dbufs[b][r, sl]

            pltpu.async_copy(dbufs[b], out_slice(j), sems_w[b])

    @pl.loop(0, G_CPW_PAD, step=2)
    def _(j):
        step(j, 0)
        step(j + 1, 1)

    pltpu.make_async_copy(dbufs[0], out_slice(0), sems_w[0]).wait()


def _edge_body(gw_ref, aw_ref, gm_ref, am_ref,
               cw_ref, b1w_ref, w2w_ref, b2w_ref,
               cm_ref, b1m_ref, w2m_ref, b2m_ref,
               ew_ref, em_ref):
    hw = (gw_ref[...]
          + jnp.dot(aw_ref[...], cw_ref[...],
                    preferred_element_type=jnp.float32) + b1w_ref[...])
    hw = jnp.maximum(hw, 0.0)
    ew_ref[...] = (jnp.dot(hw, w2w_ref[...],
                           preferred_element_type=jnp.float32) + b2w_ref[...])
    hm = (gm_ref[...]
          + jnp.dot(am_ref[...], cm_ref[...],
                    preferred_element_type=jnp.float32) + b1m_ref[...])
    hm = jnp.maximum(hm, 0.0)
    em_ref[...] = (jnp.dot(hm, w2m_ref[...],
                           preferred_element_type=jnp.float32) + b2m_ref[...])


def _edge_mlp(g_cat, attrw, attrm, cw, b1w, w2w, b2w, cm, b1m, w2m, b2m):
    e_tile = 512
    grid = (N_EDGES // e_tile,)
    n_tiles = N_EDGES // e_tile
    gw_spec = pl.BlockSpec((e_tile, D_FEAT), lambda t: (t, 0))
    gm_spec = pl.BlockSpec((e_tile, D_FEAT), lambda t: (t + n_tiles, 0))
    g_spec = pl.BlockSpec((e_tile, D_FEAT), lambda t: (t, 0))
    a_spec = pl.BlockSpec((e_tile, D_EDGE), lambda t: (t, 0))
    c_spec = pl.BlockSpec((D_EDGE, D_FEAT), lambda t: (0, 0))
    w_spec = pl.BlockSpec((D_FEAT, D_FEAT), lambda t: (0, 0))
    b_spec = pl.BlockSpec((1, D_FEAT), lambda t: (0, 0))
    out = jax.ShapeDtypeStruct((N_EDGES, D_FEAT), jnp.float32)
    return pl.pallas_call(
        _edge_body,
        grid=grid,
        in_specs=[gw_spec, a_spec, gm_spec, a_spec,
                  c_spec, b_spec, w_spec, b_spec,
                  c_spec, b_spec, w_spec, b_spec],
        out_specs=[g_spec, g_spec],
        out_shape=[out, out],
    )(g_cat, attrw, g_cat, attrm, cw, b1w, w2w, b2w, cm, b1m, w2m, b2m)


@functools.partial(
    pl.kernel,
    out_type=[jax.ShapeDtypeStruct((N_NODES, D_FEAT), jnp.float32),
              jax.ShapeDtypeStruct((N_NODES, D_FEAT), jnp.float32)],
    mesh=_VMESH,
    scratch_types=[
        pltpu.VMEM_SHARED((N_NODES, D_FEAT), jnp.float32),
        pltpu.VMEM((ROW_CHUNK, D_FEAT), jnp.float32),
        pltpu.VMEM((CHUNK,), jnp.int32),
        pltpu.VMEM((CHUNK, D_FEAT), jnp.float32),
    ],
)
def _sc_segsum(ew_hbm, em_hbm, dstw_hbm, dstm_hbm,
               aw_hbm, am_hbm,
               agg_sh, zbuf, idx_v, ebuf):
    cid = lax.axis_index("c")
    sid = lax.axis_index("s")

    @pl.loop(0, ROW_CHUNK)
    def _(r):
        @pl.loop(0, D_FEAT, step=16)
        def _(cc):
            zbuf[r, pl.ds(cc, 16)] = jnp.zeros((16,), jnp.float32)

    @pl.loop(0, ROW_CPS)
    def _(k):
        rchunk = sid + NS * k

        @pl.when(rchunk < N_ROW_CHUNKS)
        def _():
            rbase = pl.multiple_of(rchunk * ROW_CHUNK, 8)
            pltpu.sync_copy(zbuf, agg_sh.at[pl.ds(rbase, ROW_CHUNK)])

    plsc.subcore_barrier()

    def accumulate(e_hbm, dst_hbm):
        @pl.loop(0, SCAT_CPS)
        def _(j):
            chunk = sid + NS * j

            @pl.when(chunk < N_CHUNKS)
            def _():
                base = pl.multiple_of(chunk * CHUNK, 8)
                pltpu.sync_copy(dst_hbm.at[pl.ds(base, CHUNK)], idx_v)
                pltpu.sync_copy(e_hbm.at[pl.ds(base, CHUNK)], ebuf)
                pltpu.sync_copy(ebuf, agg_sh.at[idx_v], add=True)

    @pl.when(cid == 0)
    def _():
        accumulate(ew_hbm, dstw_hbm)

    @pl.when(cid == 1)
    def _():
        accumulate(em_hbm, dstm_hbm)

    plsc.subcore_barrier()

    def copy_out(out_hbm):
        @pl.loop(0, ROW_CPS)
        def _(k):
            rchunk = sid + NS * k

            @pl.when(rchunk < N_ROW_CHUNKS)
            def _():
                rbase = pl.multiple_of(rchunk * ROW_CHUNK, 8)
                sl = pl.ds(rbase, ROW_CHUNK)
                pltpu.sync_copy(agg_sh.at[sl], out_hbm.at[sl])

    @pl.when(cid == 0)
    def _():
        copy_out(aw_hbm)

    @pl.when(cid == 1)
    def _():
        copy_out(am_hbm)


def _node_body(x_ref, aw_ref, am_ref, wn1_ref, bn1_ref, wn2_ref, bn2_ref,
               out_ref):
    hn = jnp.concatenate([x_ref[...], aw_ref[...], am_ref[...]], axis=1)
    h = jnp.dot(hn, wn1_ref[...], preferred_element_type=jnp.float32)
    h = jnp.maximum(h + bn1_ref[...], 0.0)
    out_ref[...] = (jnp.dot(h, wn2_ref[...],
                            preferred_element_type=jnp.float32) + bn2_ref[...])


def _node_mlp(x_m, aggw, aggm, wn1, bn1, wn2, bn2):
    n_tile = 1000
    grid = (N_NODES // n_tile,)
    node_spec = pl.BlockSpec((n_tile, D_FEAT), lambda t: (t, 0))
    wn1_spec = pl.BlockSpec((3 * D_FEAT, D_FEAT), lambda t: (0, 0))
    w_spec = pl.BlockSpec((D_FEAT, D_FEAT), lambda t: (0, 0))
    b_spec = pl.BlockSpec((1, D_FEAT), lambda t: (0, 0))
    out = jax.ShapeDtypeStruct((N_NODES, D_FEAT), jnp.float32)
    return pl.pallas_call(
        _node_body,
        grid=grid,
        in_specs=[node_spec, node_spec, node_spec,
                  wn1_spec, b_spec, w_spec, b_spec],
        out_specs=node_spec,
        out_shape=out,
    )(x_m, aggw, aggm, wn1, bn1, wn2, bn2)


def kernel(x_m, x_w, edge_w, edge_m, edge_attrw, edge_attrm,
           W1w, b1w, W2w, b2w, W1m, b1m, W2m, b2m,
           Wn1, bn1, Wn2, bn2):
    srcw = edge_w[0].astype(jnp.int32)
    dstw = edge_w[1].astype(jnp.int32)
    srcm = edge_m[0].astype(jnp.int32)
    dstm = edge_m[1].astype(jnp.int32)

    wcat_w = jnp.concatenate([W1w[:D_FEAT], W1w[D_FEAT:2 * D_FEAT]], axis=1)
    wcat_m = jnp.concatenate([W1m[:D_FEAT], W1m[D_FEAT:2 * D_FEAT]], axis=1)
    cw = W1w[2 * D_FEAT:]
    cm = W1m[2 * D_FEAT:]

    x_cat = jnp.concatenate([x_w, x_m], axis=0)
    w_stack = jnp.stack([wcat_w, wcat_m], axis=0)
    s_cat, d_cat = _premix(x_cat, w_stack)

    pad = IDX_PAD - 2 * N_EDGES
    isrc = jnp.pad(jnp.concatenate([srcw, srcm + N_NODES]), (0, pad))
    idst = jnp.pad(jnp.concatenate([dstw, dstm + N_NODES]), (0, pad))

    g_cat = _sc_gather(s_cat, d_cat, isrc, idst)

    ew, em = _edge_mlp(g_cat, edge_attrw, edge_attrm,
                       cw, b1w.reshape(1, -1), W2w, b2w.reshape(1, -1),
                       cm, b1m.reshape(1, -1), W2m, b2m.reshape(1, -1))

    aggw, aggm = _sc_segsum(ew, em, dstw, dstm)

    x = _node_mlp(x_m, aggw, aggm, Wn1,
                  bn1.reshape(1, -1), Wn2, bn2.reshape(1, -1))
    return (x, ew, em)


# speedup vs baseline: 3.1973x; 1.1098x over previous
import functools

import jax
import jax.numpy as jnp
from jax import lax
from jax.experimental import pallas as pl
from jax.experimental.pallas import tpu as pltpu
from jax.experimental.pallas import tpu_sc as plsc

N_NODES = 10000
N_EDGES = 320000
D_FEAT = 128
D_EDGE = 16

NC, NS = 2, 16
NW = NC * NS
CHUNK = 128
N_CHUNKS = N_EDGES // CHUNK
SCAT_CPS = 160
SCAT_PAD_CHUNKS = NS * SCAT_CPS

G_CHUNKS = 2 * N_CHUNKS
G_CPW = -(-G_CHUNKS // NW)
G_CPW_PAD = G_CPW + (G_CPW % 2)
IDX_SPAN = G_CPW_PAD * CHUNK
IDX_PAD = (NW - 1) * G_CPW * CHUNK + IDX_SPAN
IDX_RING = 16
ROW_CHUNK = 40
N_ROW_CHUNKS = N_NODES // ROW_CHUNK
ROW_CPS = -(-N_ROW_CHUNKS // NS)

_VMESH = plsc.VectorSubcoreMesh(
    core_axis_name="c", subcore_axis_name="s", num_cores=NC, num_subcores=NS)


def _premix_body(x_ref, w_ref, s_ref, d_ref):
    p = jnp.dot(x_ref[...], w_ref[0], preferred_element_type=jnp.float32)
    s_ref[...] = p[:, :D_FEAT]
    d_ref[...] = p[:, D_FEAT:]


def _premix(x_cat, w_stack):
    n_tile = 1000
    grid = (2 * N_NODES // n_tile,)
    node_spec = pl.BlockSpec((n_tile, D_FEAT), lambda t: (t, 0))
    w_spec = pl.BlockSpec((1, D_FEAT, 2 * D_FEAT), lambda t: (t // 10, 0, 0))
    out = jax.ShapeDtypeStruct((2 * N_NODES, D_FEAT), jnp.float32)
    return pl.pallas_call(
        _premix_body,
        grid=grid,
        in_specs=[node_spec, w_spec],
        out_specs=[node_spec] * 2,
        out_shape=[out] * 2,
    )(x_cat, w_stack)


@functools.partial(
    pl.kernel,
    out_type=jax.ShapeDtypeStruct((2 * N_EDGES, D_FEAT), jnp.float32),
    mesh=_VMESH,
    scratch_types=[
        pltpu.VMEM((IDX_SPAN,), jnp.int32),
        pltpu.VMEM((IDX_SPAN,), jnp.int32),
        [pltpu.VMEM((CHUNK, D_FEAT), jnp.float32)] * 2,
        [pltpu.VMEM((CHUNK, D_FEAT), jnp.float32)] * 2,
        [pltpu.SemaphoreType.DMA] * 2,
        [pltpu.SemaphoreType.DMA] * 2,
    ],
)
def _sc_gather(s_tab, d_tab, isrc_hbm, idst_hbm, out_hbm,
               ibuf_s, ibuf_d, sbufs, dbufs, sems_g, sems_w):
    wid = lax.axis_index("s") * NC + lax.axis_index("c")
    span0 = pl.multiple_of(wid * (G_CPW * CHUNK), 8)

    pltpu.sync_copy(isrc_hbm.at[pl.ds(span0, IDX_SPAN)], ibuf_s)
    pltpu.sync_copy(idst_hbm.at[pl.ds(span0, IDX_SPAN)], ibuf_d)

    def rvalid(j):
        return jnp.logical_and(j < G_CPW, wid * G_CPW + j < G_CHUNKS)

    def issue_gather(j, b):
        isl = pl.ds(pl.multiple_of(j * CHUNK, 8), CHUNK)
        pltpu.async_copy(s_tab.at[ibuf_s.at[isl]], sbufs[b], sems_g[b])
        pltpu.async_copy(d_tab.at[ibuf_d.at[isl]], dbufs[b], sems_g[b])

    def drain_gather(j, b):
        isl = pl.ds(pl.multiple_of(j * CHUNK, 8), CHUNK)
        pltpu.make_async_copy(s_tab.at[ibuf_s.at[isl]], sbufs[b],
                              sems_g[b]).wait()
        pltpu.make_async_copy(d_tab.at[ibuf_d.at[isl]], dbufs[b],
                              sems_g[b]).wait()

    def out_slice(j):
        base = pl.multiple_of((wid * G_CPW + j) * CHUNK, 8)
        return out_hbm.at[pl.ds(base, CHUNK)]

    issue_gather(0, 0)

    def step(j, b):
        @pl.when(rvalid(j))
        def _():
            drain_gather(j, b)

            @pl.when(jnp.logical_and(j >= 1, rvalid(j - 1)))
            def _():
                pltpu.make_async_copy(dbufs[1 - b], out_slice(j - 1),
                                      sems_w[1 - b]).wait()

            @pl.when(rvalid(j + 1))
            def _():
                issue_gather(j + 1, 1 - b)

            @pl.loop(0, CHUNK)
            def _(r):
                for cc in range(0, D_FEAT, 16):
                    sl = pl.ds(cc, 16)
                    dbufs[b][r, sl] = sbufs[b][r, sl] + dbufs[b][r, sl]

            pltpu.async_copy(dbufs[b], out_slice(j), sems_w[b])

    @pl.loop(0, G_CPW_PAD, step=2)
    def _(j):
        step(j, 0)
        step(j + 1, 1)

    pltpu.make_async_copy(dbufs[0], out_slice(0), sems_w[0]).wait()


def _edge_body(gw_ref, aw_ref, gm_ref, am_ref,
               cw_ref, b1w_ref, w2w_ref, b2w_ref,
               cm_ref, b1m_ref, w2m_ref, b2m_ref,
               ew_ref, em_ref):
    hw = (gw_ref[...]
          + jnp.dot(aw_ref[...], cw_ref[...],
                    preferred_element_type=jnp.float32) + b1w_ref[...])
    hw = jnp.maximum(hw, 0.0)
    ew_ref[...] = (jnp.dot(hw, w2w_ref[...],
                           preferred_element_type=jnp.float32) + b2w_ref[...])
    hm = (gm_ref[...]
          + jnp.dot(am_ref[...], cm_ref[...],
                    preferred_element_type=jnp.float32) + b1m_ref[...])
    hm = jnp.maximum(hm, 0.0)
    em_ref[...] = (jnp.dot(hm, w2m_ref[...],
                           preferred_element_type=jnp.float32) + b2m_ref[...])


def _edge_mlp(g_cat, attrw, attrm, cw, b1w, w2w, b2w, cm, b1m, w2m, b2m):
    e_tile = 512
    grid = (N_EDGES // e_tile,)
    n_tiles = N_EDGES // e_tile
    gw_spec = pl.BlockSpec((e_tile, D_FEAT), lambda t: (t, 0))
    gm_spec = pl.BlockSpec((e_tile, D_FEAT), lambda t: (t + n_tiles, 0))
    g_spec = pl.BlockSpec((e_tile, D_FEAT), lambda t: (t, 0))
    a_spec = pl.BlockSpec((e_tile, D_EDGE), lambda t: (t, 0))
    c_spec = pl.BlockSpec((D_EDGE, D_FEAT), lambda t: (0, 0))
    w_spec = pl.BlockSpec((D_FEAT, D_FEAT), lambda t: (0, 0))
    b_spec = pl.BlockSpec((1, D_FEAT), lambda t: (0, 0))
    out = jax.ShapeDtypeStruct((N_EDGES, D_FEAT), jnp.float32)
    return pl.pallas_call(
        _edge_body,
        grid=grid,
        in_specs=[gw_spec, a_spec, gm_spec, a_spec,
                  c_spec, b_spec, w_spec, b_spec,
                  c_spec, b_spec, w_spec, b_spec],
        out_specs=[g_spec, g_spec],
        out_shape=[out, out],
    )(g_cat, attrw, g_cat, attrm, cw, b1w, w2w, b2w, cm, b1m, w2m, b2m)


@functools.partial(
    pl.kernel,
    out_type=[jax.ShapeDtypeStruct((N_NODES, D_FEAT), jnp.float32),
              jax.ShapeDtypeStruct((N_NODES, D_FEAT), jnp.float32)],
    mesh=_VMESH,
    scratch_types=[
        pltpu.VMEM_SHARED((N_NODES, D_FEAT), jnp.float32),
        pltpu.VMEM((ROW_CHUNK, D_FEAT), jnp.float32),
        pltpu.VMEM((IDX_RING, CHUNK), jnp.int32),
        [pltpu.VMEM((CHUNK, D_FEAT), jnp.float32)] * 2,
        [pltpu.SemaphoreType.DMA] * 2,
    ],
)
def _sc_segsum(ew_hbm, em_hbm, dstw_hbm, dstm_hbm,
               aw_hbm, am_hbm,
               agg_sh, zbuf, ibuf, ebufs, sems):
    cid = lax.axis_index("c")
    sid = lax.axis_index("s")
    crow0 = pl.multiple_of(sid * SCAT_CPS, 8)

    def accumulate(e_hbm, dst2d_hbm):
        def valid(j):
            return jnp.logical_and(j < SCAT_CPS,
                                   sid * SCAT_CPS + j < N_CHUNKS)

        def issue_load(j, b):
            base = pl.multiple_of((sid * SCAT_CPS + j) * CHUNK, 8)
            pltpu.async_copy(e_hbm.at[pl.ds(base, CHUNK)], ebufs[b], sems[b])

        def drain_load(j, b):
            base = pl.multiple_of((sid * SCAT_CPS + j) * CHUNK, 8)
            pltpu.make_async_copy(e_hbm.at[pl.ds(base, CHUNK)], ebufs[b],
                                  sems[b]).wait()

        issue_load(0, 0)

        @pl.loop(0, ROW_CHUNK)
        def _(r):
            @pl.loop(0, D_FEAT, step=16)
            def _(cc):
                zbuf[r, pl.ds(cc, 16)] = jnp.zeros((16,), jnp.float32)

        @pl.loop(0, ROW_CPS)
        def _(k):
            rchunk = sid + NS * k

            @pl.when(rchunk < N_ROW_CHUNKS)
            def _():
                rbase = pl.multiple_of(rchunk * ROW_CHUNK, 8)
                pltpu.sync_copy(zbuf, agg_sh.at[pl.ds(rbase, ROW_CHUNK)])

        plsc.subcore_barrier()

        def step(j, b):
            @pl.when(valid(j))
            def _():
                @pl.when(lax.rem(j, IDX_RING) == 0)
                def _():
                    pltpu.sync_copy(
                        dst2d_hbm.at[pl.ds(pl.multiple_of(crow0 + j, 8),
                                           IDX_RING)], ibuf)

                drain_load(j, b)

                @pl.when(valid(j + 1))
                def _():
                    issue_load(j + 1, 1 - b)

                pltpu.sync_copy(ebufs[b],
                                agg_sh.at[ibuf.at[lax.rem(j, IDX_RING)]],
                                add=True)

        @pl.loop(0, SCAT_CPS, step=2)
        def _(j):
            step(j, 0)
            step(j + 1, 1)

    @pl.when(cid == 0)
    def _():
        accumulate(ew_hbm, dstw_hbm)

    @pl.when(cid == 1)
    def _():
        accumulate(em_hbm, dstm_hbm)

    plsc.subcore_barrier()

    def copy_out(out_hbm):
        @pl.loop(0, ROW_CPS)
        def _(k):
            rchunk = sid + NS * k

            @pl.when(rchunk < N_ROW_CHUNKS)
            def _():
                rbase = pl.multiple_of(rchunk * ROW_CHUNK, 8)
                sl = pl.ds(rbase, ROW_CHUNK)
                pltpu.sync_copy(agg_sh.at[sl], out_hbm.at[sl])

    @pl.when(cid == 0)
    def _():
        copy_out(aw_hbm)

    @pl.when(cid == 1)
    def _():
        copy_out(am_hbm)


def _node_body(x_ref, aw_ref, am_ref, wn1_ref, bn1_ref, wn2_ref, bn2_ref,
               out_ref):
    hn = jnp.concatenate([x_ref[...], aw_ref[...], am_ref[...]], axis=1)
    h = jnp.dot(hn, wn1_ref[...], preferred_element_type=jnp.float32)
    h = jnp.maximum(h + bn1_ref[...], 0.0)
    out_ref[...] = (jnp.dot(h, wn2_ref[...],
                            preferred_element_type=jnp.float32) + bn2_ref[...])


def _node_mlp(x_m, aggw, aggm, wn1, bn1, wn2, bn2):
    n_tile = 1000
    grid = (N_NODES // n_tile,)
    node_spec = pl.BlockSpec((n_tile, D_FEAT), lambda t: (t, 0))
    wn1_spec = pl.BlockSpec((3 * D_FEAT, D_FEAT), lambda t: (0, 0))
    w_spec = pl.BlockSpec((D_FEAT, D_FEAT), lambda t: (0, 0))
    b_spec = pl.BlockSpec((1, D_FEAT), lambda t: (0, 0))
    out = jax.ShapeDtypeStruct((N_NODES, D_FEAT), jnp.float32)
    return pl.pallas_call(
        _node_body,
        grid=grid,
        in_specs=[node_spec, node_spec, node_spec,
                  wn1_spec, b_spec, w_spec, b_spec],
        out_specs=node_spec,
        out_shape=out,
    )(x_m, aggw, aggm, wn1, bn1, wn2, bn2)


def kernel(x_m, x_w, edge_w, edge_m, edge_attrw, edge_attrm,
           W1w, b1w, W2w, b2w, W1m, b1m, W2m, b2m,
           Wn1, bn1, Wn2, bn2):
    srcw = edge_w[0].astype(jnp.int32)
    dstw = edge_w[1].astype(jnp.int32)
    srcm = edge_m[0].astype(jnp.int32)
    dstm = edge_m[1].astype(jnp.int32)

    wcat_w = jnp.concatenate([W1w[:D_FEAT], W1w[D_FEAT:2 * D_FEAT]], axis=1)
    wcat_m = jnp.concatenate([W1m[:D_FEAT], W1m[D_FEAT:2 * D_FEAT]], axis=1)
    cw = W1w[2 * D_FEAT:]
    cm = W1m[2 * D_FEAT:]

    x_cat = jnp.concatenate([x_w, x_m], axis=0)
    w_stack = jnp.stack([wcat_w, wcat_m], axis=0)
    s_cat, d_cat = _premix(x_cat, w_stack)

    pad = IDX_PAD - 2 * N_EDGES
    isrc = jnp.pad(jnp.concatenate([srcw, srcm + N_NODES]), (0, pad))
    idst = jnp.pad(jnp.concatenate([dstw, dstm + N_NODES]), (0, pad))

    g_cat = _sc_gather(s_cat, d_cat, isrc, idst)

    ew, em = _edge_mlp(g_cat, edge_attrw, edge_attrm,
                       cw, b1w.reshape(1, -1), W2w, b2w.reshape(1, -1),
                       cm, b1m.reshape(1, -1), W2m, b2m.reshape(1, -1))

    spad = SCAT_PAD_CHUNKS * CHUNK - N_EDGES
    dstw2d = jnp.pad(dstw, (0, spad)).reshape(SCAT_PAD_CHUNKS, CHUNK)
    dstm2d = jnp.pad(dstm, (0, spad)).reshape(SCAT_PAD_CHUNKS, CHUNK)

    aggw, aggm = _sc_segsum(ew, em, dstw2d, dstm2d)

    x = _node_mlp(x_m, aggw, aggm, Wn1,
                  bn1.reshape(1, -1), Wn2, bn2.reshape(1, -1))
    return (x, ew, em)


# speedup vs baseline: 3.6688x; 1.1475x over previous
import functools

import jax
import jax.numpy as jnp
from jax import lax
from jax.experimental import pallas as pl
from jax.experimental.pallas import tpu as pltpu
from jax.experimental.pallas import tpu_sc as plsc

N_NODES = 10000
N_EDGES = 320000
D_FEAT = 128
D_EDGE = 16

NC, NS = 2, 16
NW = NC * NS
CHUNK = 128
N_CHUNKS = N_EDGES // CHUNK
SCAT_CPS = 160
SCAT_PAD_CHUNKS = NS * SCAT_CPS

G_CHUNKS = 2 * N_CHUNKS
G_CPW = -(-G_CHUNKS // NW)
G_CPW_PAD = G_CPW + (G_CPW % 2)
IDX_SPAN = G_CPW_PAD * CHUNK
IDX_PAD = (NW - 1) * G_CPW * CHUNK + IDX_SPAN
IDX_RING = 16
ROW_CHUNK = 40
N_ROW_CHUNKS = N_NODES // ROW_CHUNK
ROW_CPS = -(-N_ROW_CHUNKS // NS)

_VMESH = plsc.VectorSubcoreMesh(
    core_axis_name="c", subcore_axis_name="s", num_cores=NC, num_subcores=NS)


def _premix_body(x_ref, w_ref, s_ref, d_ref):
    p = jnp.dot(x_ref[...].astype(jnp.bfloat16), w_ref[0],
                preferred_element_type=jnp.float32)
    s_ref[...] = p[:, :D_FEAT]
    d_ref[...] = p[:, D_FEAT:]


def _premix(x_cat, w_stack):
    n_tile = 1000
    grid = (2 * N_NODES // n_tile,)
    node_spec = pl.BlockSpec((n_tile, D_FEAT), lambda t: (t, 0))
    w_spec = pl.BlockSpec((1, D_FEAT, 2 * D_FEAT), lambda t: (t // 10, 0, 0))
    out = jax.ShapeDtypeStruct((2 * N_NODES, D_FEAT), jnp.float32)
    return pl.pallas_call(
        _premix_body,
        grid=grid,
        in_specs=[node_spec, w_spec],
        out_specs=[node_spec] * 2,
        out_shape=[out] * 2,
    )(x_cat, w_stack)


@functools.partial(
    pl.kernel,
    out_type=jax.ShapeDtypeStruct((2 * N_EDGES, D_FEAT), jnp.float32),
    mesh=_VMESH,
    scratch_types=[
        pltpu.VMEM((IDX_SPAN,), jnp.int32),
        pltpu.VMEM((IDX_SPAN,), jnp.int32),
        [pltpu.VMEM((CHUNK, D_FEAT), jnp.float32)] * 2,
        [pltpu.VMEM((CHUNK, D_FEAT), jnp.float32)] * 2,
        [pltpu.SemaphoreType.DMA] * 2,
        [pltpu.SemaphoreType.DMA] * 2,
    ],
)
def _sc_gather(s_tab, d_tab, isrc_hbm, idst_hbm, out_hbm,
               ibuf_s, ibuf_d, sbufs, dbufs, sems_g, sems_w):
    wid = lax.axis_index("s") * NC + lax.axis_index("c")
    span0 = pl.multiple_of(wid * (G_CPW * CHUNK), 8)

    pltpu.sync_copy(isrc_hbm.at[pl.ds(span0, IDX_SPAN)], ibuf_s)
    pltpu.sync_copy(idst_hbm.at[pl.ds(span0, IDX_SPAN)], ibuf_d)

    def rvalid(j):
        return jnp.logical_and(j < G_CPW, wid * G_CPW + j < G_CHUNKS)

    def issue_gather(j, b):
        isl = pl.ds(pl.multiple_of(j * CHUNK, 8), CHUNK)
        pltpu.async_copy(s_tab.at[ibuf_s.at[isl]], sbufs[b], sems_g[b])
        pltpu.async_copy(d_tab.at[ibuf_d.at[isl]], dbufs[b], sems_g[b])

    def drain_gather(j, b):
        isl = pl.ds(pl.multiple_of(j * CHUNK, 8), CHUNK)
        pltpu.make_async_copy(s_tab.at[ibuf_s.at[isl]], sbufs[b],
                              sems_g[b]).wait()
        pltpu.make_async_copy(d_tab.at[ibuf_d.at[isl]], dbufs[b],
                              sems_g[b]).wait()

    def out_slice(j):
        base = pl.multiple_of((wid * G_CPW + j) * CHUNK, 8)
        return out_hbm.at[pl.ds(base, CHUNK)]

    issue_gather(0, 0)

    def step(j, b):
        @pl.when(rvalid(j))
        def _():
            drain_gather(j, b)

            @pl.when(jnp.logical_and(j >= 1, rvalid(j - 1)))
            def _():
                pltpu.make_async_copy(dbufs[1 - b], out_slice(j - 1),
                                      sems_w[1 - b]).wait()

            @pl.when(rvalid(j + 1))
            def _():
                issue_gather(j + 1, 1 - b)

            @pl.loop(0, CHUNK)
            def _(r):
                for cc in range(0, D_FEAT, 16):
                    sl = pl.ds(cc, 16)
                    dbufs[b][r, sl] = sbufs[b][r, sl] + dbufs[b][r, sl]

            pltpu.async_copy(dbufs[b], out_slice(j), sems_w[b])

    @pl.loop(0, G_CPW_PAD, step=2)
    def _(j):
        step(j, 0)
        step(j + 1, 1)

    pltpu.make_async_copy(dbufs[0], out_slice(0), sems_w[0]).wait()


def _edge_body(gw_ref, aw_ref, gm_ref, am_ref,
               cw_ref, b1w_ref, w2w_ref, b2w_ref,
               cm_ref, b1m_ref, w2m_ref, b2m_ref,
               ew_ref, em_ref):
    hw = (gw_ref[...]
          + jnp.dot(aw_ref[...].astype(jnp.bfloat16), cw_ref[...],
                    preferred_element_type=jnp.float32) + b1w_ref[...])
    hw = jnp.maximum(hw, 0.0).astype(jnp.bfloat16)
    ew_ref[...] = (jnp.dot(hw, w2w_ref[...],
                           preferred_element_type=jnp.float32) + b2w_ref[...])
    hm = (gm_ref[...]
          + jnp.dot(am_ref[...].astype(jnp.bfloat16), cm_ref[...],
                    preferred_element_type=jnp.float32) + b1m_ref[...])
    hm = jnp.maximum(hm, 0.0).astype(jnp.bfloat16)
    em_ref[...] = (jnp.dot(hm, w2m_ref[...],
                           preferred_element_type=jnp.float32) + b2m_ref[...])


def _edge_mlp(g_cat, attrw, attrm, cw, b1w, w2w, b2w, cm, b1m, w2m, b2m):
    e_tile = 1000
    grid = (N_EDGES // e_tile,)
    n_tiles = N_EDGES // e_tile
    gw_spec = pl.BlockSpec((e_tile, D_FEAT), lambda t: (t, 0))
    gm_spec = pl.BlockSpec((e_tile, D_FEAT), lambda t: (t + n_tiles, 0))
    g_spec = pl.BlockSpec((e_tile, D_FEAT), lambda t: (t, 0))
    a_spec = pl.BlockSpec((e_tile, D_EDGE), lambda t: (t, 0))
    c_spec = pl.BlockSpec((D_EDGE, D_FEAT), lambda t: (0, 0))
    w_spec = pl.BlockSpec((D_FEAT, D_FEAT), lambda t: (0, 0))
    b_spec = pl.BlockSpec((1, D_FEAT), lambda t: (0, 0))
    out = jax.ShapeDtypeStruct((N_EDGES, D_FEAT), jnp.float32)
    return pl.pallas_call(
        _edge_body,
        grid=grid,
        in_specs=[gw_spec, a_spec, gm_spec, a_spec,
                  c_spec, b_spec, w_spec, b_spec,
                  c_spec, b_spec, w_spec, b_spec],
        out_specs=[g_spec, g_spec],
        out_shape=[out, out],
    )(g_cat, attrw, g_cat, attrm, cw, b1w, w2w, b2w, cm, b1m, w2m, b2m)


@functools.partial(
    pl.kernel,
    out_type=[jax.ShapeDtypeStruct((N_NODES, D_FEAT), jnp.float32),
              jax.ShapeDtypeStruct((N_NODES, D_FEAT), jnp.float32)],
    mesh=_VMESH,
    scratch_types=[
        pltpu.VMEM_SHARED((N_NODES, D_FEAT), jnp.float32),
        pltpu.VMEM((ROW_CHUNK, D_FEAT), jnp.float32),
        pltpu.VMEM((IDX_RING, CHUNK), jnp.int32),
        [pltpu.VMEM((CHUNK, D_FEAT), jnp.float32)] * 2,
        [pltpu.SemaphoreType.DMA] * 2,
    ],
)
def _sc_segsum(ew_hbm, em_hbm, dstw_hbm, dstm_hbm,
               aw_hbm, am_hbm,
               agg_sh, zbuf, ibuf, ebufs, sems):
    cid = lax.axis_index("c")
    sid = lax.axis_index("s")
    crow0 = pl.multiple_of(sid * SCAT_CPS, 8)

    def accumulate(e_hbm, dst2d_hbm):
        def valid(j):
            return jnp.logical_and(j < SCAT_CPS,
                                   sid * SCAT_CPS + j < N_CHUNKS)

        def issue_load(j, b):
            base = pl.multiple_of((sid * SCAT_CPS + j) * CHUNK, 8)
            pltpu.async_copy(e_hbm.at[pl.ds(base, CHUNK)], ebufs[b], sems[b])

        def drain_load(j, b):
            base = pl.multiple_of((sid * SCAT_CPS + j) * CHUNK, 8)
            pltpu.make_async_copy(e_hbm.at[pl.ds(base, CHUNK)], ebufs[b],
                                  sems[b]).wait()

        issue_load(0, 0)

        @pl.loop(0, ROW_CHUNK)
        def _(r):
            @pl.loop(0, D_FEAT, step=16)
            def _(cc):
                zbuf[r, pl.ds(cc, 16)] = jnp.zeros((16,), jnp.float32)

        @pl.loop(0, ROW_CPS)
        def _(k):
            rchunk = sid + NS * k

            @pl.when(rchunk < N_ROW_CHUNKS)
            def _():
                rbase = pl.multiple_of(rchunk * ROW_CHUNK, 8)
                pltpu.sync_copy(zbuf, agg_sh.at[pl.ds(rbase, ROW_CHUNK)])

        plsc.subcore_barrier()

        def step(j, b):
            @pl.when(valid(j))
            def _():
                @pl.when(lax.rem(j, IDX_RING) == 0)
                def _():
                    pltpu.sync_copy(
                        dst2d_hbm.at[pl.ds(pl.multiple_of(crow0 + j, 8),
                                           IDX_RING)], ibuf)

                drain_load(j, b)

                @pl.when(valid(j + 1))
                def _():
                    issue_load(j + 1, 1 - b)

                pltpu.sync_copy(ebufs[b],
                                agg_sh.at[ibuf.at[lax.rem(j, IDX_RING)]],
                                add=True)

        @pl.loop(0, SCAT_CPS, step=2)
        def _(j):
            step(j, 0)
            step(j + 1, 1)

    @pl.when(cid == 0)
    def _():
        accumulate(ew_hbm, dstw_hbm)

    @pl.when(cid == 1)
    def _():
        accumulate(em_hbm, dstm_hbm)

    plsc.subcore_barrier()

    def copy_out(out_hbm):
        @pl.loop(0, ROW_CPS)
        def _(k):
            rchunk = sid + NS * k

            @pl.when(rchunk < N_ROW_CHUNKS)
            def _():
                rbase = pl.multiple_of(rchunk * ROW_CHUNK, 8)
                sl = pl.ds(rbase, ROW_CHUNK)
                pltpu.sync_copy(agg_sh.at[sl], out_hbm.at[sl])

    @pl.when(cid == 0)
    def _():
        copy_out(aw_hbm)

    @pl.when(cid == 1)
    def _():
        copy_out(am_hbm)


def _node_body(x_ref, aw_ref, am_ref, wn1_ref, bn1_ref, wn2_ref, bn2_ref,
               out_ref):
    hn = jnp.concatenate([x_ref[...], aw_ref[...], am_ref[...]],
                         axis=1).astype(jnp.bfloat16)
    h = jnp.dot(hn, wn1_ref[...], preferred_element_type=jnp.float32)
    h = jnp.maximum(h + bn1_ref[...], 0.0).astype(jnp.bfloat16)
    out_ref[...] = (jnp.dot(h, wn2_ref[...],
                            preferred_element_type=jnp.float32) + bn2_ref[...])


def _node_mlp(x_m, aggw, aggm, wn1, bn1, wn2, bn2):
    n_tile = 1000
    grid = (N_NODES // n_tile,)
    node_spec = pl.BlockSpec((n_tile, D_FEAT), lambda t: (t, 0))
    wn1_spec = pl.BlockSpec((3 * D_FEAT, D_FEAT), lambda t: (0, 0))
    w_spec = pl.BlockSpec((D_FEAT, D_FEAT), lambda t: (0, 0))
    b_spec = pl.BlockSpec((1, D_FEAT), lambda t: (0, 0))
    out = jax.ShapeDtypeStruct((N_NODES, D_FEAT), jnp.float32)
    return pl.pallas_call(
        _node_body,
        grid=grid,
        in_specs=[node_spec, node_spec, node_spec,
                  wn1_spec, b_spec, w_spec, b_spec],
        out_specs=node_spec,
        out_shape=out,
    )(x_m, aggw, aggm, wn1, bn1, wn2, bn2)


def kernel(x_m, x_w, edge_w, edge_m, edge_attrw, edge_attrm,
           W1w, b1w, W2w, b2w, W1m, b1m, W2m, b2m,
           Wn1, bn1, Wn2, bn2):
    srcw = edge_w[0].astype(jnp.int32)
    dstw = edge_w[1].astype(jnp.int32)
    srcm = edge_m[0].astype(jnp.int32)
    dstm = edge_m[1].astype(jnp.int32)

    wcat_w = jnp.concatenate([W1w[:D_FEAT], W1w[D_FEAT:2 * D_FEAT]], axis=1)
    wcat_m = jnp.concatenate([W1m[:D_FEAT], W1m[D_FEAT:2 * D_FEAT]], axis=1)
    cw = W1w[2 * D_FEAT:]
    cm = W1m[2 * D_FEAT:]

    x_cat = jnp.concatenate([x_w, x_m], axis=0)
    w_stack = jnp.stack([wcat_w, wcat_m], axis=0).astype(jnp.bfloat16)
    s_cat, d_cat = _premix(x_cat, w_stack)

    pad = IDX_PAD - 2 * N_EDGES
    isrc = jnp.pad(jnp.concatenate([srcw, srcm + N_NODES]), (0, pad))
    idst = jnp.pad(jnp.concatenate([dstw, dstm + N_NODES]), (0, pad))

    g_cat = _sc_gather(s_cat, d_cat, isrc, idst)

    bf = jnp.bfloat16
    ew, em = _edge_mlp(g_cat, edge_attrw, edge_attrm,
                       cw.astype(bf), b1w.reshape(1, -1),
                       W2w.astype(bf), b2w.reshape(1, -1),
                       cm.astype(bf), b1m.reshape(1, -1),
                       W2m.astype(bf), b2m.reshape(1, -1))

    spad = SCAT_PAD_CHUNKS * CHUNK - N_EDGES
    dstw2d = jnp.pad(dstw, (0, spad)).reshape(SCAT_PAD_CHUNKS, CHUNK)
    dstm2d = jnp.pad(dstm, (0, spad)).reshape(SCAT_PAD_CHUNKS, CHUNK)

    aggw, aggm = _sc_segsum(ew, em, dstw2d, dstm2d)

    x = _node_mlp(x_m, aggw, aggm, Wn1.astype(bf),
                  bn1.reshape(1, -1), Wn2.astype(bf), bn2.reshape(1, -1))
    return (x, ew, em)
